# Initial kernel scaffold; baseline (speedup 1.0000x reference)
import jax
import jax.numpy as jnp
from jax.experimental import pallas as pl


def kernel(ids, weight):
    raise NotImplementedError("write your pallas kernel here")


# speedup vs baseline: 1.0938x; 1.0938x over previous
import jax
import jax.numpy as jnp
from jax import lax
from jax.experimental import pallas as pl
from jax.experimental.pallas import tpu as pltpu
from jax.experimental.pallas import tpu_sc as plsc

NC = 2
NS = 16
NW = NC * NS

CHUNK = 1024


def _emb_body(ids_hbm, table_hbm, out_hbm, idx_v, rows_v, sem):
    n_chunks = ids_hbm.shape[0] // (NW * CHUNK)
    b_per_w = n_chunks * CHUNK
    wid = lax.axis_index("s") * NC + lax.axis_index("c")
    base = wid * b_per_w

    def chunk_body(i, carry):
        off = base + i * CHUNK
        pltpu.sync_copy(ids_hbm.at[pl.ds(off, CHUNK)], idx_v)
        pltpu.async_copy(table_hbm.at[idx_v], rows_v, sem).wait()
        pltpu.sync_copy(rows_v, out_hbm.at[pl.ds(off, CHUNK)])
        return carry

    lax.fori_loop(0, n_chunks, chunk_body, 0)


def kernel(ids, weight):
    orig_shape = ids.shape
    dim = weight.shape[1]
    flat = ids.reshape(-1).astype(jnp.int32)
    total = flat.shape[0]
    pad = (-total) % (NW * CHUNK)
    if pad:
        flat = jnp.concatenate([flat, jnp.zeros((pad,), jnp.int32)])

    mesh = plsc.VectorSubcoreMesh(core_axis_name="c", subcore_axis_name="s")
    out = pl.kernel(
        _emb_body,
        out_type=jax.ShapeDtypeStruct((total + pad, dim), jnp.float32),
        mesh=mesh,
        compiler_params=pltpu.CompilerParams(use_tc_tiling_on_sc=False),
        scratch_types=[
            pltpu.VMEM((CHUNK,), jnp.int32),
            pltpu.VMEM((CHUNK, dim), jnp.float32),
            pltpu.SemaphoreType.DMA,
        ],
    )(flat, weight)
    if pad:
        out = out[:total]
    return out.reshape(*orig_shape, dim)
